# baseline (speedup 1.0000x reference)
import jax
import jax.numpy as jnp
from jax.experimental import pallas as pl

N_ROWS = 100000
EMB = 256
HID = 128
OUT = 3
BLK = 2048


def _body(n_ref, c_ref, r_ref, wn1_ref, wn2_ref, wr1_ref, wr2_ref,
          bn_ref, br_ref, wd1_ref, wd2_ref, bd_ref, out_ref):
    c = c_ref[...]
    nc = jnp.dot(n_ref[...], wn1_ref[...], preferred_element_type=jnp.float32)
    nc += jnp.dot(c, wn2_ref[...], preferred_element_type=jnp.float32)
    nc = jnp.maximum(nc + bn_ref[...], 0.0)
    rc = jnp.dot(r_ref[...], wr1_ref[...], preferred_element_type=jnp.float32)
    rc += jnp.dot(c, wr2_ref[...], preferred_element_type=jnp.float32)
    rc = jnp.maximum(rc + br_ref[...], 0.0)
    out = jnp.dot(nc, wd1_ref[...], preferred_element_type=jnp.float32)
    out += jnp.dot(rc, wd2_ref[...], preferred_element_type=jnp.float32)
    out_ref[...] = out + bd_ref[...]


def kernel(neighbor, current, remote, W_n, b_n, W_r, b_r, W_d, b_d):
    grid = (pl.cdiv(N_ROWS, BLK),)
    row_spec = pl.BlockSpec((BLK, EMB), lambda i: (i, 0))
    full = lambda shape: pl.BlockSpec(shape, lambda i: (0, 0))
    out = pl.pallas_call(
        _body,
        grid=grid,
        in_specs=[
            row_spec, row_spec, row_spec,
            full((EMB, HID)), full((EMB, HID)),
            full((EMB, HID)), full((EMB, HID)),
            full((1, HID)), full((1, HID)),
            full((HID, OUT)), full((HID, OUT)),
            full((1, OUT)),
        ],
        out_specs=pl.BlockSpec((BLK, OUT), lambda i: (i, 0)),
        out_shape=jax.ShapeDtypeStruct((N_ROWS, OUT), jnp.float32),
    )(
        neighbor, current, remote,
        W_n[:EMB], W_n[EMB:], W_r[:EMB], W_r[EMB:],
        b_n.reshape(1, HID), b_r.reshape(1, HID),
        W_d[:HID], W_d[HID:], b_d.reshape(1, OUT),
    )
    return out
